# baseline (speedup 1.0000x reference)
import jax
import jax.numpy as jnp
from jax import lax
from jax.experimental import pallas as pl
from jax.experimental.pallas import tpu as pltpu
from jax.experimental.pallas import tpu_sc as plsc

HIDDEN = 64
LANES = 16
NC, NS = 2, 16
NW = NC * NS
CH = 128
SCALE = 8.0


def _build(total):
    assert total % (NW * CH) == 0
    G = total // (NW * CH)
    mesh = plsc.VectorSubcoreMesh(
        core_axis_name="c", subcore_axis_name="s",
        num_cores=NC, num_subcores=NS)

    def body(idx_hbm, table_hbm, out_hbm, idx_v, rows_v, gsem):
        wid = lax.axis_index("s") * NC + lax.axis_index("c")
        base = wid * (G * CH)
        pltpu.sync_copy(idx_hbm.at[wid], idx_v)

        @pl.loop(0, G)
        def _chunk(g):
            pltpu.async_copy(table_hbm.at[idx_v.at[g]], rows_v, gsem).wait()

            @pl.loop(0, CH)
            def _row(r):
                for j in range(HIDDEN // LANES):
                    sl = pl.ds(j * LANES, LANES)
                    rows_v[r, sl] = rows_v[r, sl] * SCALE

            pltpu.sync_copy(rows_v, out_hbm.at[pl.ds(base + g * CH, CH)])

    return pl.kernel(
        body,
        out_type=jax.ShapeDtypeStruct((total, HIDDEN), jnp.float32),
        mesh=mesh,
        scratch_types=[
            pltpu.VMEM((G, CH), jnp.int32),
            pltpu.VMEM((CH, HIDDEN), jnp.float32),
            pltpu.SemaphoreType.DMA,
        ],
        compiler_params=pltpu.CompilerParams(use_tc_tiling_on_sc=False),
    )


def kernel(x, table):
    b, s = x.shape
    total = b * s
    idx = x.reshape(NW, total // (NW * CH), CH).astype(jnp.int32)
    out = _build(total)(idx, table)
    return out.reshape(b, s, HIDDEN)


# speedup vs baseline: 1.2119x; 1.2119x over previous
import jax
import jax.numpy as jnp
from jax import lax
from jax.experimental import pallas as pl
from jax.experimental.pallas import tpu as pltpu
from jax.experimental.pallas import tpu_sc as plsc

HIDDEN = 64
LANES = 16
NC, NS = 2, 16
NW = NC * NS
CH = 128
NB = 4
SCALE = 8.0


def _build(total):
    assert total % (NW * CH) == 0
    G = total // (NW * CH)
    assert G % NB == 0
    steps = G // NB
    mesh = plsc.VectorSubcoreMesh(
        core_axis_name="c", subcore_axis_name="s",
        num_cores=NC, num_subcores=NS)

    def body(idx_hbm, table_hbm, out_hbm, idx_v, gbuf, sbuf, gsem, ssem):
        wid = lax.axis_index("s") * NC + lax.axis_index("c")
        base = wid * (G * CH)
        pltpu.sync_copy(idx_hbm.at[wid], idx_v)

        for b in range(NB):
            pltpu.async_copy(table_hbm.at[idx_v.at[b]], gbuf.at[b], gsem.at[b])

        @pl.loop(0, steps)
        def _step(step):
            for b in range(NB):
                g = step * NB + b
                pltpu.make_async_copy(
                    table_hbm.at[idx_v.at[g]], gbuf.at[b], gsem.at[b]).wait()

                @pl.when(step > 0)
                def _():
                    pltpu.make_async_copy(
                        sbuf.at[b], out_hbm.at[pl.ds(base, CH)],
                        ssem.at[b]).wait()

                @pl.loop(0, CH)
                def _row(r):
                    for j in range(HIDDEN // LANES):
                        sl = pl.ds(j * LANES, LANES)
                        sbuf[b, r, sl] = gbuf[b, r, sl] * SCALE

                @pl.when(step < steps - 1)
                def _():
                    pltpu.async_copy(
                        table_hbm.at[idx_v.at[g + NB]], gbuf.at[b],
                        gsem.at[b])

                pltpu.async_copy(
                    sbuf.at[b], out_hbm.at[pl.ds(base + g * CH, CH)],
                    ssem.at[b])

        for b in range(NB):
            pltpu.make_async_copy(
                sbuf.at[b], out_hbm.at[pl.ds(base, CH)], ssem.at[b]).wait()

    return pl.kernel(
        body,
        out_type=jax.ShapeDtypeStruct((total, HIDDEN), jnp.float32),
        mesh=mesh,
        scratch_types=[
            pltpu.VMEM((G, CH), jnp.int32),
            pltpu.VMEM((NB, CH, HIDDEN), jnp.float32),
            pltpu.VMEM((NB, CH, HIDDEN), jnp.float32),
            pltpu.SemaphoreType.DMA((NB,)),
            pltpu.SemaphoreType.DMA((NB,)),
        ],
        compiler_params=pltpu.CompilerParams(use_tc_tiling_on_sc=False),
    )


def kernel(x, table):
    b, s = x.shape
    total = b * s
    idx = x.reshape(NW, total // (NW * CH), CH).astype(jnp.int32)
    out = _build(total)(idx, table)
    return out.reshape(b, s, HIDDEN)
